# Initial kernel scaffold; baseline (speedup 1.0000x reference)
import jax
import jax.numpy as jnp
from jax.experimental import pallas as pl


def kernel(rpn_scores, rpn_reg, xyz):
    raise NotImplementedError("write your pallas kernel here")


# speedup vs baseline: 80.6576x; 80.6576x over previous
import functools

import jax
import jax.numpy as jnp
import numpy as np
from jax.experimental import pallas as pl
from jax.experimental.pallas import tpu as pltpu

_NMS_POST = 512
_NMS_THRES = 0.85
_LOC_SCOPE = 3.0
_LOC_BIN_SIZE = 0.5
_NUM_HEAD_BIN = 12
_MEAN_SIZE = (1.53, 1.63, 3.88)
_NEG = float(np.float32(-np.inf))


def _decode_body(reg_ref, xyz_ref, prop_ref, bev_ref):
    reg = reg_ref[...]
    lane = jax.lax.broadcasted_iota(jnp.int32, reg.shape, 1)
    p = int(_LOC_SCOPE / _LOC_BIN_SIZE) * 2

    def seg_argmax(lo, width):
        m = jnp.where((lane >= lo) & (lane < lo + width), reg, _NEG)
        mx = jnp.max(m, axis=1, keepdims=True)
        b = jnp.min(jnp.where(m == mx, lane, 1 << 30), axis=1, keepdims=True)
        return b - lo

    def gather_lane(abs_lane):
        return jnp.sum(jnp.where(lane == abs_lane, reg, 0.0), axis=1,
                       keepdims=True)

    x_bin = seg_argmax(0, p)
    z_bin = seg_argmax(p, p)
    x_res = gather_lane(2 * p + x_bin)
    z_res = gather_lane(3 * p + z_bin)

    xb = x_bin.astype(jnp.float32)
    zb = z_bin.astype(jnp.float32)
    pos_x = xb * _LOC_BIN_SIZE + (_LOC_BIN_SIZE / 2.0) - _LOC_SCOPE
    pos_z = zb * _LOC_BIN_SIZE + (_LOC_BIN_SIZE / 2.0) - _LOC_SCOPE
    pos_x = pos_x + x_res * _LOC_BIN_SIZE
    pos_z = pos_z + z_res * _LOC_BIN_SIZE

    cx = xyz_ref[:, 0:1]
    cy = xyz_ref[:, 1:2]
    cz = xyz_ref[:, 2:3]

    start = 4 * p
    y_off = gather_lane(jnp.full_like(x_bin, start))
    pos_y = cy + y_off

    ry_bin = seg_argmax(start + 1, _NUM_HEAD_BIN)
    ry_res_norm = gather_lane(start + 1 + _NUM_HEAD_BIN + ry_bin)
    angle_per_class = 2.0 * np.pi / _NUM_HEAD_BIN
    ry_res = ry_res_norm * (angle_per_class / 2.0)
    ry = ry_bin.astype(jnp.float32) * angle_per_class + ry_res
    two_pi = np.float32(2.0 * np.pi)
    r = jax.lax.rem(ry, jnp.full_like(ry, two_pi))
    ry = jnp.where(r < 0.0, r + two_pi, r)
    ry = jnp.where(ry > np.pi, ry - two_pi, ry)

    s0 = start + 1 + 2 * _NUM_HEAD_BIN
    h = gather_lane(jnp.full_like(x_bin, s0 + 0)) * _MEAN_SIZE[0] + _MEAN_SIZE[0]
    w = gather_lane(jnp.full_like(x_bin, s0 + 1)) * _MEAN_SIZE[1] + _MEAN_SIZE[1]
    l = gather_lane(jnp.full_like(x_bin, s0 + 2)) * _MEAN_SIZE[2] + _MEAN_SIZE[2]

    pos_x = pos_x + cx
    pos_z = pos_z + cz

    lane8 = jax.lax.broadcasted_iota(jnp.int32, (reg.shape[0], 8), 1)
    zero = jnp.zeros_like(pos_x)

    def pack8(cols):
        acc = jnp.zeros((reg.shape[0], 8), jnp.float32)
        for c, v in enumerate(cols):
            acc = acc + jnp.where(lane8 == c, v, 0.0)
        return acc

    prop_ref[...] = pack8([pos_x, pos_y, pos_z, h, w, l, ry, zero])

    half_w = h / 2.0
    half_h = l / 2.0
    bev_ref[...] = pack8([pos_x - half_w, pos_z - half_h,
                          pos_x + half_w, pos_z + half_h, zero, zero, zero,
                          zero])


def _nms_body(x1_ref, z1_ref, x2_ref, z2_ref, sc_ref, prop_ref, out_ref, *,
              n_valid):
    x1 = x1_ref[0]
    z1 = z1_ref[0]
    x2 = x2_ref[0]
    z2 = z2_ref[0]
    sc = sc_ref[0]
    rows, lanes = x1.shape
    i0 = jax.lax.broadcasted_iota(jnp.int32, (rows, lanes), 0)
    i1 = jax.lax.broadcasted_iota(jnp.int32, (rows, lanes), 1)
    pos = i0 * lanes + i1
    areas = (x2 - x1) * (z2 - z1)
    lane8 = jax.lax.broadcasted_iota(jnp.int32, (1, 8), 1)

    def body(k, alive):
        masked = jnp.where(alive > 0.0, sc, _NEG)
        m = jnp.max(masked)
        valid = m > _NEG
        idx = jnp.min(jnp.where(masked == m, pos, 1 << 30))
        idx = jnp.where(valid, idx, 0)
        sel = pos == idx
        x1i = jnp.sum(jnp.where(sel, x1, 0.0))
        z1i = jnp.sum(jnp.where(sel, z1, 0.0))
        x2i = jnp.sum(jnp.where(sel, x2, 0.0))
        z2i = jnp.sum(jnp.where(sel, z2, 0.0))
        area_i = (x2i - x1i) * (z2i - z1i)

        xx1 = jnp.maximum(x1i, x1)
        zz1 = jnp.maximum(z1i, z1)
        xx2 = jnp.minimum(x2i, x2)
        zz2 = jnp.minimum(z2i, z2)
        inter = jnp.maximum(xx2 - xx1, 0.0) * jnp.maximum(zz2 - zz1, 0.0)
        iou = inter / jnp.maximum(area_i + areas - inter, 1e-8)
        sup = (iou > _NMS_THRES) | sel
        new_alive = jnp.where(sup, 0.0, alive)
        alive = jnp.where(valid, new_alive, alive)

        vf = valid.astype(jnp.float32)
        prow = prop_ref[0, pl.ds(idx, 1), :]
        row = (prow + jnp.where(lane8 == 7, m, 0.0)) * vf
        out_ref[0, pl.ds(k, 1), :] = row
        return alive

    alive0 = (pos < n_valid).astype(jnp.float32)
    jax.lax.fori_loop(0, _NMS_POST, body, alive0)


def kernel(rpn_scores, rpn_reg, xyz):
    B, N = rpn_scores.shape
    C = rpn_reg.shape[-1]
    BN = B * N
    reg = rpn_reg.reshape(BN, C)
    xyz8 = jnp.pad(xyz.reshape(BN, 3), ((0, 0), (0, 5)))

    blk = 2000 if BN % 2000 == 0 else BN
    grid = BN // blk
    prop, bev = pl.pallas_call(
        _decode_body,
        grid=(grid,),
        in_specs=[
            pl.BlockSpec((blk, C), lambda i: (i, 0)),
            pl.BlockSpec((blk, 8), lambda i: (i, 0)),
        ],
        out_specs=[
            pl.BlockSpec((blk, 8), lambda i: (i, 0)),
            pl.BlockSpec((blk, 8), lambda i: (i, 0)),
        ],
        out_shape=[
            jax.ShapeDtypeStruct((BN, 8), jnp.float32),
            jax.ShapeDtypeStruct((BN, 8), jnp.float32),
        ],
    )(reg, xyz8)

    NP = ((N + 127) // 128) * 128
    rows = NP // 128

    def chan(c):
        a = bev[:, c].reshape(B, N)
        return jnp.pad(a, ((0, 0), (0, NP - N))).reshape(B, rows, 128)

    x1, z1, x2, z2 = chan(0), chan(1), chan(2), chan(3)
    scp = jnp.pad(rpn_scores, ((0, 0), (0, NP - N))).reshape(B, rows, 128)
    prop_p = jnp.pad(prop.reshape(B, N, 8), ((0, 0), (0, NP - N), (0, 0)))

    vspec = pl.BlockSpec((1, rows, 128), lambda b: (b, 0, 0))
    out = pl.pallas_call(
        functools.partial(_nms_body, n_valid=N),
        grid=(B,),
        in_specs=[vspec, vspec, vspec, vspec, vspec,
                  pl.BlockSpec((1, NP, 8), lambda b: (b, 0, 0))],
        out_specs=pl.BlockSpec((1, _NMS_POST, 8), lambda b: (b, 0, 0)),
        out_shape=jax.ShapeDtypeStruct((B, _NMS_POST, 8), jnp.float32),
    )(x1, z1, x2, z2, scp, prop_p)

    return out[..., :7], out[..., 7]


# speedup vs baseline: 101.8622x; 1.2629x over previous
import functools

import jax
import jax.numpy as jnp
import numpy as np
from jax.experimental import pallas as pl
from jax.experimental.pallas import tpu as pltpu

_NMS_POST = 512
_NMS_THRES = 0.85
_LOC_SCOPE = 3.0
_LOC_BIN_SIZE = 0.5
_NUM_HEAD_BIN = 12
_MEAN_SIZE = (1.53, 1.63, 3.88)
_NEG = float(np.float32(-np.inf))


def _decode_body(reg_ref, xyz_ref, prop_ref, bev_ref):
    reg = reg_ref[...]
    lane = jax.lax.broadcasted_iota(jnp.int32, reg.shape, 1)
    p = int(_LOC_SCOPE / _LOC_BIN_SIZE) * 2

    def seg_argmax(lo, width):
        m = jnp.where((lane >= lo) & (lane < lo + width), reg, _NEG)
        mx = jnp.max(m, axis=1, keepdims=True)
        b = jnp.min(jnp.where(m == mx, lane, 1 << 30), axis=1, keepdims=True)
        return b - lo

    def gather_lane(abs_lane):
        return jnp.sum(jnp.where(lane == abs_lane, reg, 0.0), axis=1,
                       keepdims=True)

    x_bin = seg_argmax(0, p)
    z_bin = seg_argmax(p, p)
    x_res = gather_lane(2 * p + x_bin)
    z_res = gather_lane(3 * p + z_bin)

    xb = x_bin.astype(jnp.float32)
    zb = z_bin.astype(jnp.float32)
    pos_x = xb * _LOC_BIN_SIZE + (_LOC_BIN_SIZE / 2.0) - _LOC_SCOPE
    pos_z = zb * _LOC_BIN_SIZE + (_LOC_BIN_SIZE / 2.0) - _LOC_SCOPE
    pos_x = pos_x + x_res * _LOC_BIN_SIZE
    pos_z = pos_z + z_res * _LOC_BIN_SIZE

    cx = xyz_ref[:, 0:1]
    cy = xyz_ref[:, 1:2]
    cz = xyz_ref[:, 2:3]

    start = 4 * p
    y_off = gather_lane(jnp.full_like(x_bin, start))
    pos_y = cy + y_off

    ry_bin = seg_argmax(start + 1, _NUM_HEAD_BIN)
    ry_res_norm = gather_lane(start + 1 + _NUM_HEAD_BIN + ry_bin)
    angle_per_class = 2.0 * np.pi / _NUM_HEAD_BIN
    ry_res = ry_res_norm * (angle_per_class / 2.0)
    ry = ry_bin.astype(jnp.float32) * angle_per_class + ry_res
    two_pi = np.float32(2.0 * np.pi)
    r = jax.lax.rem(ry, jnp.full_like(ry, two_pi))
    ry = jnp.where(r < 0.0, r + two_pi, r)
    ry = jnp.where(ry > np.pi, ry - two_pi, ry)

    s0 = start + 1 + 2 * _NUM_HEAD_BIN
    h = gather_lane(jnp.full_like(x_bin, s0 + 0)) * _MEAN_SIZE[0] + _MEAN_SIZE[0]
    w = gather_lane(jnp.full_like(x_bin, s0 + 1)) * _MEAN_SIZE[1] + _MEAN_SIZE[1]
    l = gather_lane(jnp.full_like(x_bin, s0 + 2)) * _MEAN_SIZE[2] + _MEAN_SIZE[2]

    pos_x = pos_x + cx
    pos_z = pos_z + cz

    lane8 = jax.lax.broadcasted_iota(jnp.int32, (reg.shape[0], 8), 1)
    zero = jnp.zeros_like(pos_x)

    def pack8(cols):
        acc = jnp.zeros((reg.shape[0], 8), jnp.float32)
        for c, v in enumerate(cols):
            acc = acc + jnp.where(lane8 == c, v, 0.0)
        return acc

    prop_ref[...] = pack8([pos_x, pos_y, pos_z, h, w, l, ry, zero])

    half_w = h / 2.0
    half_h = l / 2.0
    bev_ref[...] = pack8([pos_x - half_w, pos_z - half_h,
                          pos_x + half_w, pos_z + half_h, zero, zero, zero,
                          zero])


def _nms_body(x1_ref, z1_ref, x2_ref, z2_ref, sc_ref, bev_ref, prop_ref,
              out_ref, *, n_valid, nb):
    x1 = [x1_ref[b] for b in range(nb)]
    z1 = [z1_ref[b] for b in range(nb)]
    x2 = [x2_ref[b] for b in range(nb)]
    z2 = [z2_ref[b] for b in range(nb)]
    sc = [sc_ref[b] for b in range(nb)]
    rows, lanes = x1[0].shape
    i0 = jax.lax.broadcasted_iota(jnp.int32, (rows, lanes), 0)
    i1 = jax.lax.broadcasted_iota(jnp.int32, (rows, lanes), 1)
    pos = i0 * lanes + i1
    areas = [(x2[b] - x1[b]) * (z2[b] - z1[b]) for b in range(nb)]
    lane8 = jax.lax.broadcasted_iota(jnp.int32, (1, 8), 1)

    def body(k, alive):
        out = []
        for b in range(nb):
            masked = jnp.where(alive[b] > 0.0, sc[b], _NEG)
            m = jnp.max(masked)
            valid = m > _NEG
            idx = jnp.min(jnp.where(masked == m, pos, 1 << 30))
            idx = jnp.where(valid, idx, 0)
            sel = pos == idx
            brow = bev_ref[b, pl.ds(idx, 1), :]
            x1i = brow[0, 0]
            z1i = brow[0, 1]
            x2i = brow[0, 2]
            z2i = brow[0, 3]
            area_i = (x2i - x1i) * (z2i - z1i)

            xx1 = jnp.maximum(x1i, x1[b])
            zz1 = jnp.maximum(z1i, z1[b])
            xx2 = jnp.minimum(x2i, x2[b])
            zz2 = jnp.minimum(z2i, z2[b])
            inter = jnp.maximum(xx2 - xx1, 0.0) * jnp.maximum(zz2 - zz1, 0.0)
            iou = inter / jnp.maximum(area_i + areas[b] - inter, 1e-8)
            sup = (iou > _NMS_THRES) | sel
            new_alive = jnp.where(sup, 0.0, alive[b])
            out.append(jnp.where(valid, new_alive, alive[b]))

            vf = valid.astype(jnp.float32)
            prow = prop_ref[b, pl.ds(idx, 1), :]
            row = (prow + jnp.where(lane8 == 7, m, 0.0)) * vf
            out_ref[b, pl.ds(k, 1), :] = row
        return out

    alive0 = [(pos < n_valid).astype(jnp.float32) for _ in range(nb)]
    jax.lax.fori_loop(0, _NMS_POST, body, alive0)


def kernel(rpn_scores, rpn_reg, xyz):
    B, N = rpn_scores.shape
    C = rpn_reg.shape[-1]
    BN = B * N
    reg = rpn_reg.reshape(BN, C)
    xyz8 = jnp.pad(xyz.reshape(BN, 3), ((0, 0), (0, 5)))

    blk = 2000 if BN % 2000 == 0 else BN
    grid = BN // blk
    prop, bev = pl.pallas_call(
        _decode_body,
        grid=(grid,),
        in_specs=[
            pl.BlockSpec((blk, C), lambda i: (i, 0)),
            pl.BlockSpec((blk, 8), lambda i: (i, 0)),
        ],
        out_specs=[
            pl.BlockSpec((blk, 8), lambda i: (i, 0)),
            pl.BlockSpec((blk, 8), lambda i: (i, 0)),
        ],
        out_shape=[
            jax.ShapeDtypeStruct((BN, 8), jnp.float32),
            jax.ShapeDtypeStruct((BN, 8), jnp.float32),
        ],
    )(reg, xyz8)

    NP = ((N + 127) // 128) * 128
    rows = NP // 128

    def chan(c):
        a = bev[:, c].reshape(B, N)
        return jnp.pad(a, ((0, 0), (0, NP - N))).reshape(B, rows, 128)

    x1, z1, x2, z2 = chan(0), chan(1), chan(2), chan(3)
    scp = jnp.pad(rpn_scores, ((0, 0), (0, NP - N))).reshape(B, rows, 128)
    bev_p = jnp.pad(bev.reshape(B, N, 8), ((0, 0), (0, NP - N), (0, 0)))
    prop_p = jnp.pad(prop.reshape(B, N, 8), ((0, 0), (0, NP - N), (0, 0)))

    out = pl.pallas_call(
        functools.partial(_nms_body, n_valid=N, nb=B),
        out_shape=jax.ShapeDtypeStruct((B, _NMS_POST, 8), jnp.float32),
    )(x1, z1, x2, z2, scp, bev_p, prop_p)

    return out[..., :7], out[..., 7]


# speedup vs baseline: 155.0412x; 1.5221x over previous
import functools

import jax
import jax.numpy as jnp
import numpy as np
from jax.experimental import pallas as pl
from jax.experimental.pallas import tpu as pltpu

_NMS_POST = 512
_NMS_THRES = 0.85
_LOC_SCOPE = 3.0
_LOC_BIN_SIZE = 0.5
_NUM_HEAD_BIN = 12
_MEAN_SIZE = (1.53, 1.63, 3.88)
_NEG = float(np.float32(-np.inf))


def _decode_body(reg_ref, xyz_ref, prop_ref, bev_ref):
    reg = reg_ref[...]
    lane = jax.lax.broadcasted_iota(jnp.int32, reg.shape, 1)
    p = int(_LOC_SCOPE / _LOC_BIN_SIZE) * 2

    def seg_argmax(lo, width):
        m = jnp.where((lane >= lo) & (lane < lo + width), reg, _NEG)
        mx = jnp.max(m, axis=1, keepdims=True)
        b = jnp.min(jnp.where(m == mx, lane, 1 << 30), axis=1, keepdims=True)
        return b - lo

    def gather_lane(abs_lane):
        return jnp.sum(jnp.where(lane == abs_lane, reg, 0.0), axis=1,
                       keepdims=True)

    x_bin = seg_argmax(0, p)
    z_bin = seg_argmax(p, p)
    x_res = gather_lane(2 * p + x_bin)
    z_res = gather_lane(3 * p + z_bin)

    xb = x_bin.astype(jnp.float32)
    zb = z_bin.astype(jnp.float32)
    pos_x = xb * _LOC_BIN_SIZE + (_LOC_BIN_SIZE / 2.0) - _LOC_SCOPE
    pos_z = zb * _LOC_BIN_SIZE + (_LOC_BIN_SIZE / 2.0) - _LOC_SCOPE
    pos_x = pos_x + x_res * _LOC_BIN_SIZE
    pos_z = pos_z + z_res * _LOC_BIN_SIZE

    cx = xyz_ref[:, 0:1]
    cy = xyz_ref[:, 1:2]
    cz = xyz_ref[:, 2:3]

    start = 4 * p
    y_off = gather_lane(jnp.full_like(x_bin, start))
    pos_y = cy + y_off

    ry_bin = seg_argmax(start + 1, _NUM_HEAD_BIN)
    ry_res_norm = gather_lane(start + 1 + _NUM_HEAD_BIN + ry_bin)
    angle_per_class = 2.0 * np.pi / _NUM_HEAD_BIN
    ry_res = ry_res_norm * (angle_per_class / 2.0)
    ry = ry_bin.astype(jnp.float32) * angle_per_class + ry_res
    two_pi = np.float32(2.0 * np.pi)
    r = jax.lax.rem(ry, jnp.full_like(ry, two_pi))
    ry = jnp.where(r < 0.0, r + two_pi, r)
    ry = jnp.where(ry > np.pi, ry - two_pi, ry)

    s0 = start + 1 + 2 * _NUM_HEAD_BIN
    h = gather_lane(jnp.full_like(x_bin, s0 + 0)) * _MEAN_SIZE[0] + _MEAN_SIZE[0]
    w = gather_lane(jnp.full_like(x_bin, s0 + 1)) * _MEAN_SIZE[1] + _MEAN_SIZE[1]
    l = gather_lane(jnp.full_like(x_bin, s0 + 2)) * _MEAN_SIZE[2] + _MEAN_SIZE[2]

    pos_x = pos_x + cx
    pos_z = pos_z + cz

    lane8 = jax.lax.broadcasted_iota(jnp.int32, (reg.shape[0], 8), 1)
    zero = jnp.zeros_like(pos_x)

    def pack8(cols):
        acc = jnp.zeros((reg.shape[0], 8), jnp.float32)
        for c, v in enumerate(cols):
            acc = acc + jnp.where(lane8 == c, v, 0.0)
        return acc

    prop_ref[...] = pack8([pos_x, pos_y, pos_z, h, w, l, ry, zero])

    half_w = h / 2.0
    half_h = l / 2.0
    bev_ref[...] = pack8([pos_x - half_w, pos_z - half_h,
                          pos_x + half_w, pos_z + half_h, zero, zero, zero,
                          zero])


def _nms_body(x1_ref, z1_ref, x2_ref, z2_ref, sc_ref, bev_ref, prop_ref,
              out_ref, *, n_valid, nb):
    x1 = x1_ref[...]
    z1 = z1_ref[...]
    x2 = x2_ref[...]
    z2 = z2_ref[...]
    sc = sc_ref[...]
    V = x1.shape[0]
    iv = jax.lax.broadcasted_iota(jnp.int32, x1.shape, 0)
    isub = jax.lax.broadcasted_iota(jnp.int32, x1.shape, 1)
    il = jax.lax.broadcasted_iota(jnp.int32, x1.shape, 2)
    sh = 8 // nb
    pos = iv * (sh * 128) + (isub // nb) * 128 + il
    areas = (x2 - x1) * (z2 - z1)
    lane8 = jax.lax.broadcasted_iota(jnp.int32, (1, 8), 1)
    sub8 = jax.lax.broadcasted_iota(jnp.int32, (8, 1), 0)
    scene8 = sub8 % nb

    def col_from_scalars(vals):
        acc = jnp.full((8, 1), vals[-1], jnp.float32)
        for b in range(nb - 1):
            acc = jnp.where(scene8 == b, vals[b], acc)
        return acc

    def body(k, alive):
        masked = jnp.where(alive > 0.0, sc, _NEG)
        r1 = jnp.max(masked, axis=0)
        r2 = jnp.maximum(r1, pltpu.roll(r1, nb, 0))
        mcol = jnp.max(r2, axis=1, keepdims=True)
        cand = jnp.where(masked == mcol[None], pos, 1 << 30)
        i1 = jnp.min(cand, axis=0)
        i2 = jnp.minimum(i1, pltpu.roll(i1, nb, 0))
        icol = jnp.min(i2, axis=1, keepdims=True)
        validc = mcol > _NEG

        sel = pos == icol[None]
        vcol3 = validc[None]

        x1s, z1s, x2s, z2s, ms, vfs, idxs = [], [], [], [], [], [], []
        for b in range(nb):
            m_b = mcol[b, 0]
            valid_b = m_b > _NEG
            idx_b = jnp.where(valid_b, icol[b, 0], 0)
            brow = bev_ref[b, pl.ds(idx_b, 1), :]
            x1s.append(brow[0, 0])
            z1s.append(brow[0, 1])
            x2s.append(brow[0, 2])
            z2s.append(brow[0, 3])
            ms.append(m_b)
            vfs.append(valid_b.astype(jnp.float32))
            idxs.append(idx_b)

        x1c = col_from_scalars(x1s)
        z1c = col_from_scalars(z1s)
        x2c = col_from_scalars(x2s)
        z2c = col_from_scalars(z2s)
        areac = (x2c - x1c) * (z2c - z1c)

        xx1 = jnp.maximum(x1c[None], x1)
        zz1 = jnp.maximum(z1c[None], z1)
        xx2 = jnp.minimum(x2c[None], x2)
        zz2 = jnp.minimum(z2c[None], z2)
        inter = jnp.maximum(xx2 - xx1, 0.0) * jnp.maximum(zz2 - zz1, 0.0)
        iou = inter / jnp.maximum(areac[None] + areas - inter, 1e-8)
        sup = (iou > _NMS_THRES) | sel
        alive = jnp.where(vcol3 & sup, 0.0, alive)

        for b in range(nb):
            prow = prop_ref[b, pl.ds(idxs[b], 1), :]
            row = (prow + jnp.where(lane8 == 7, ms[b], 0.0)) * vfs[b]
            out_ref[b, pl.ds(k, 1), :] = row
        return alive

    alive0 = (pos < n_valid).astype(jnp.float32)
    jax.lax.fori_loop(0, _NMS_POST, body, alive0)


def kernel(rpn_scores, rpn_reg, xyz):
    B, N = rpn_scores.shape
    C = rpn_reg.shape[-1]
    BN = B * N
    reg = rpn_reg.reshape(BN, C)
    xyz8 = jnp.pad(xyz.reshape(BN, 3), ((0, 0), (0, 5)))

    blk = 2000 if BN % 2000 == 0 else BN
    grid = BN // blk
    prop, bev = pl.pallas_call(
        _decode_body,
        grid=(grid,),
        in_specs=[
            pl.BlockSpec((blk, C), lambda i: (i, 0)),
            pl.BlockSpec((blk, 8), lambda i: (i, 0)),
        ],
        out_specs=[
            pl.BlockSpec((blk, 8), lambda i: (i, 0)),
            pl.BlockSpec((blk, 8), lambda i: (i, 0)),
        ],
        out_shape=[
            jax.ShapeDtypeStruct((BN, 8), jnp.float32),
            jax.ShapeDtypeStruct((BN, 8), jnp.float32),
        ],
    )(reg, xyz8)

    sh = 8 // B
    NP = ((N + sh * 128 - 1) // (sh * 128)) * (sh * 128)
    V = NP // (sh * 128)

    def interleave(a):
        a = jnp.pad(a, ((0, 0), (0, NP - N)))
        return (a.reshape(B, V, sh, 128)
                 .transpose(1, 2, 0, 3)
                 .reshape(V, 8, 128))

    x1, z1, x2, z2 = (interleave(bev[:, c].reshape(B, N)) for c in range(4))
    scp = interleave(rpn_scores)
    bev_p = jnp.pad(bev.reshape(B, N, 8), ((0, 0), (0, NP - N), (0, 0)))
    prop_p = jnp.pad(prop.reshape(B, N, 8), ((0, 0), (0, NP - N), (0, 0)))

    out = pl.pallas_call(
        functools.partial(_nms_body, n_valid=N, nb=B),
        out_shape=jax.ShapeDtypeStruct((B, _NMS_POST, 8), jnp.float32),
    )(x1, z1, x2, z2, scp, bev_p, prop_p)

    return out[..., :7], out[..., 7]


# speedup vs baseline: 226.4427x; 1.4605x over previous
import functools

import jax
import jax.numpy as jnp
import numpy as np
from jax.experimental import pallas as pl
from jax.experimental.pallas import tpu as pltpu

_NMS_POST = 512
_NMS_THRES = 0.85
_LOC_SCOPE = 3.0
_LOC_BIN_SIZE = 0.5
_NUM_HEAD_BIN = 12
_MEAN_SIZE = (1.53, 1.63, 3.88)
_NEG = float(np.float32(-np.inf))


def _decode_body(reg_ref, xyz_ref, prop_ref, bev_ref):
    reg = reg_ref[...]
    lane = jax.lax.broadcasted_iota(jnp.int32, reg.shape, 1)
    p = int(_LOC_SCOPE / _LOC_BIN_SIZE) * 2

    def seg_argmax(lo, width):
        m = jnp.where((lane >= lo) & (lane < lo + width), reg, _NEG)
        mx = jnp.max(m, axis=1, keepdims=True)
        b = jnp.min(jnp.where(m == mx, lane, 1 << 30), axis=1, keepdims=True)
        return b - lo

    def gather_lane(abs_lane):
        return jnp.sum(jnp.where(lane == abs_lane, reg, 0.0), axis=1,
                       keepdims=True)

    x_bin = seg_argmax(0, p)
    z_bin = seg_argmax(p, p)
    x_res = gather_lane(2 * p + x_bin)
    z_res = gather_lane(3 * p + z_bin)

    xb = x_bin.astype(jnp.float32)
    zb = z_bin.astype(jnp.float32)
    pos_x = xb * _LOC_BIN_SIZE + (_LOC_BIN_SIZE / 2.0) - _LOC_SCOPE
    pos_z = zb * _LOC_BIN_SIZE + (_LOC_BIN_SIZE / 2.0) - _LOC_SCOPE
    pos_x = pos_x + x_res * _LOC_BIN_SIZE
    pos_z = pos_z + z_res * _LOC_BIN_SIZE

    cx = xyz_ref[:, 0:1]
    cy = xyz_ref[:, 1:2]
    cz = xyz_ref[:, 2:3]

    start = 4 * p
    y_off = gather_lane(jnp.full_like(x_bin, start))
    pos_y = cy + y_off

    ry_bin = seg_argmax(start + 1, _NUM_HEAD_BIN)
    ry_res_norm = gather_lane(start + 1 + _NUM_HEAD_BIN + ry_bin)
    angle_per_class = 2.0 * np.pi / _NUM_HEAD_BIN
    ry_res = ry_res_norm * (angle_per_class / 2.0)
    ry = ry_bin.astype(jnp.float32) * angle_per_class + ry_res
    two_pi = np.float32(2.0 * np.pi)
    r = jax.lax.rem(ry, jnp.full_like(ry, two_pi))
    ry = jnp.where(r < 0.0, r + two_pi, r)
    ry = jnp.where(ry > np.pi, ry - two_pi, ry)

    s0 = start + 1 + 2 * _NUM_HEAD_BIN
    h = gather_lane(jnp.full_like(x_bin, s0 + 0)) * _MEAN_SIZE[0] + _MEAN_SIZE[0]
    w = gather_lane(jnp.full_like(x_bin, s0 + 1)) * _MEAN_SIZE[1] + _MEAN_SIZE[1]
    l = gather_lane(jnp.full_like(x_bin, s0 + 2)) * _MEAN_SIZE[2] + _MEAN_SIZE[2]

    pos_x = pos_x + cx
    pos_z = pos_z + cz

    lane8 = jax.lax.broadcasted_iota(jnp.int32, (reg.shape[0], 8), 1)
    zero = jnp.zeros_like(pos_x)

    def pack8(cols):
        acc = jnp.zeros((reg.shape[0], 8), jnp.float32)
        for c, v in enumerate(cols):
            acc = acc + jnp.where(lane8 == c, v, 0.0)
        return acc

    prop_ref[...] = pack8([pos_x, pos_y, pos_z, h, w, l, ry, zero])

    half_w = h / 2.0
    half_h = l / 2.0
    bev_ref[...] = pack8([pos_x - half_w, pos_z - half_h,
                          pos_x + half_w, pos_z + half_h, zero, zero, zero,
                          zero])


def _nms_body(x1_ref, z1_ref, x2_ref, z2_ref, sc_ref, bev_ref, prop_ref,
              out_ref, *, n_valid, nb):
    x1 = x1_ref[...]
    z1 = z1_ref[...]
    x2 = x2_ref[...]
    z2 = z2_ref[...]
    sc = sc_ref[...]
    V = x1.shape[0]
    iv = jax.lax.broadcasted_iota(jnp.int32, x1.shape, 0)
    isub = jax.lax.broadcasted_iota(jnp.int32, x1.shape, 1)
    il = jax.lax.broadcasted_iota(jnp.int32, x1.shape, 2)
    sh = 8 // nb
    pos = iv * (sh * 128) + (isub // nb) * 128 + il
    areas = (x2 - x1) * (z2 - z1)
    lane8 = jax.lax.broadcasted_iota(jnp.int32, (1, 8), 1)
    sub8 = jax.lax.broadcasted_iota(jnp.int32, (8, 1), 0)
    scene8 = sub8 % nb

    def shared_tree(vals, op):
        vals = list(vals)
        while len(vals) > 1:
            nxt = [op(vals[i], vals[i + 1]) for i in range(0, len(vals) - 1, 2)]
            if len(vals) % 2:
                nxt.append(vals[-1])
            vals = nxt
        r = vals[0]
        r = op(r, pltpu.roll(r, nb, 0))
        if op is jnp.maximum:
            return jnp.max(r, axis=1, keepdims=True)
        return jnp.min(r, axis=1, keepdims=True)

    def write_rows(k, icol, mcol):
        for b in range(nb):
            m_b = mcol[b, 0]
            valid_b = m_b > _NEG
            idx_b = jnp.where(valid_b, icol[b, 0], 0)
            vf = valid_b.astype(jnp.float32)
            prow = prop_ref[b, pl.ds(idx_b, 1), :]
            row = (prow + jnp.where(lane8 == 7, m_b, 0.0)) * vf
            out_ref[b, pl.ds(k, 1), :] = row

    def body(k, carry):
        alive, picol, pmcol = carry
        kk = jnp.maximum(k - 1, 0)
        write_rows(kk, picol, pmcol)

        masked = jnp.where(alive > 0.0, sc, _NEG)
        mvs = [masked[i] for i in range(V)]
        mcol = shared_tree(mvs, jnp.maximum)
        cand = jnp.where(masked == mcol[None], pos, 1 << 30)
        icol = shared_tree([cand[i] for i in range(V)], jnp.minimum)
        validc = mcol > _NEG

        sel = pos == icol[None]
        x1c = shared_tree([jnp.where(sel[i], x1[i], _NEG) for i in range(V)],
                          jnp.maximum)
        z1c = shared_tree([jnp.where(sel[i], z1[i], _NEG) for i in range(V)],
                          jnp.maximum)
        x2c = shared_tree([jnp.where(sel[i], x2[i], _NEG) for i in range(V)],
                          jnp.maximum)
        z2c = shared_tree([jnp.where(sel[i], z2[i], _NEG) for i in range(V)],
                          jnp.maximum)
        areac = (x2c - x1c) * (z2c - z1c)

        xx1 = jnp.maximum(x1c[None], x1)
        zz1 = jnp.maximum(z1c[None], z1)
        xx2 = jnp.minimum(x2c[None], x2)
        zz2 = jnp.minimum(z2c[None], z2)
        inter = jnp.maximum(xx2 - xx1, 0.0) * jnp.maximum(zz2 - zz1, 0.0)
        iou = inter / jnp.maximum(areac[None] + areas - inter, 1e-8)
        sup = (iou > _NMS_THRES) | sel
        alive = jnp.where(validc[None] & sup, 0.0, alive)
        return alive, icol, mcol

    alive0 = (pos < n_valid).astype(jnp.float32)
    icol0 = jnp.zeros((8, 1), jnp.int32)
    mcol0 = jnp.full((8, 1), _NEG, jnp.float32)
    _, icol_f, mcol_f = jax.lax.fori_loop(
        0, _NMS_POST, body, (alive0, icol0, mcol0))
    write_rows(_NMS_POST - 1, icol_f, mcol_f)


def kernel(rpn_scores, rpn_reg, xyz):
    B, N = rpn_scores.shape
    C = rpn_reg.shape[-1]
    BN = B * N
    reg = rpn_reg.reshape(BN, C)
    xyz8 = jnp.pad(xyz.reshape(BN, 3), ((0, 0), (0, 5)))

    blk = 2000 if BN % 2000 == 0 else BN
    grid = BN // blk
    prop, bev = pl.pallas_call(
        _decode_body,
        grid=(grid,),
        in_specs=[
            pl.BlockSpec((blk, C), lambda i: (i, 0)),
            pl.BlockSpec((blk, 8), lambda i: (i, 0)),
        ],
        out_specs=[
            pl.BlockSpec((blk, 8), lambda i: (i, 0)),
            pl.BlockSpec((blk, 8), lambda i: (i, 0)),
        ],
        out_shape=[
            jax.ShapeDtypeStruct((BN, 8), jnp.float32),
            jax.ShapeDtypeStruct((BN, 8), jnp.float32),
        ],
    )(reg, xyz8)

    sh = 8 // B
    NP = ((N + sh * 128 - 1) // (sh * 128)) * (sh * 128)
    V = NP // (sh * 128)

    def interleave(a):
        a = jnp.pad(a, ((0, 0), (0, NP - N)))
        return (a.reshape(B, V, sh, 128)
                 .transpose(1, 2, 0, 3)
                 .reshape(V, 8, 128))

    x1, z1, x2, z2 = (interleave(bev[:, c].reshape(B, N)) for c in range(4))
    scp = interleave(rpn_scores)
    bev_p = jnp.pad(bev.reshape(B, N, 8), ((0, 0), (0, NP - N), (0, 0)))
    prop_p = jnp.pad(prop.reshape(B, N, 8), ((0, 0), (0, NP - N), (0, 0)))

    out = pl.pallas_call(
        functools.partial(_nms_body, n_valid=N, nb=B),
        out_shape=jax.ShapeDtypeStruct((B, _NMS_POST, 8), jnp.float32),
    )(x1, z1, x2, z2, scp, bev_p, prop_p)

    return out[..., :7], out[..., 7]


# speedup vs baseline: 260.5935x; 1.1508x over previous
import functools

import jax
import jax.numpy as jnp
import numpy as np
from jax import lax
from jax.experimental import pallas as pl
from jax.experimental.pallas import tpu as pltpu
from jax.experimental.pallas import tpu_sc as plsc

_NMS_POST = 512
_NMS_THRES = 0.85
_LOC_SCOPE = 3.0
_LOC_BIN_SIZE = 0.5
_NUM_HEAD_BIN = 12
_MEAN_SIZE = (1.53, 1.63, 3.88)
_NEG = float(np.float32(-np.inf))

_NC, _NS = 2, 16
_NTILES = _NC * _NS
_PITCH = 81
_XPITCH = 5


def _sc_decode_body(reg_hbm, xyz_hbm, out_hbm, reg_v, xyz_v, out_v,
                    rows_per_tile):
    wid = lax.axis_index("s") * _NC + lax.axis_index("c")
    base = wid * rows_per_tile
    pltpu.sync_copy(reg_hbm.at[pl.ds(base * _PITCH, rows_per_tile * _PITCH)],
                    reg_v)
    pltpu.sync_copy(xyz_hbm.at[pl.ds(base * _XPITCH, rows_per_tile * _XPITCH)],
                    xyz_v)

    iota16 = lax.iota(jnp.int32, 16)
    zero16 = jnp.zeros((16,), jnp.float32)
    p = int(_LOC_SCOPE / _LOC_BIN_SIZE) * 2

    def full16(c):
        return jnp.full((16,), c, jnp.int32)

    def g_body(g, carry):
        rows = g * 16 + iota16
        rbase = rows * _PITCH
        xbase = rows * _XPITCH

        def ld(c):
            return plsc.load_gather(reg_v, [rbase + c])

        def ldx(c):
            return plsc.load_gather(xyz_v, [xbase + c])

        def seg_argmax(lo):
            m = ld(lo)
            b = jnp.zeros((16,), jnp.int32)
            for c in range(1, p):
                v = ld(lo + c)
                gt = v > m
                m = jnp.where(gt, v, m)
                b = jnp.where(gt, full16(c), b)
            return b

        x_bin = seg_argmax(0)
        z_bin = seg_argmax(p)
        x_res = plsc.load_gather(reg_v, [rbase + 2 * p + x_bin])
        z_res = plsc.load_gather(reg_v, [rbase + 3 * p + z_bin])

        xb = x_bin.astype(jnp.float32)
        zb = z_bin.astype(jnp.float32)
        pos_x = xb * _LOC_BIN_SIZE + (_LOC_BIN_SIZE / 2.0) - _LOC_SCOPE
        pos_z = zb * _LOC_BIN_SIZE + (_LOC_BIN_SIZE / 2.0) - _LOC_SCOPE
        pos_x = pos_x + x_res * _LOC_BIN_SIZE
        pos_z = pos_z + z_res * _LOC_BIN_SIZE

        cx = ldx(0)
        cy = ldx(1)
        cz = ldx(2)

        start = 4 * p
        pos_y = cy + ld(start)

        ry_bin = seg_argmax(start + 1)
        ry_res_norm = plsc.load_gather(
            reg_v, [rbase + start + 1 + _NUM_HEAD_BIN + ry_bin])
        angle_per_class = 2.0 * np.pi / _NUM_HEAD_BIN
        ry_res = ry_res_norm * (angle_per_class / 2.0)
        ry = ry_bin.astype(jnp.float32) * angle_per_class + ry_res
        two_pi = np.float32(2.0 * np.pi)
        r = lax.rem(ry, jnp.full((16,), two_pi))
        ry = jnp.where(r < 0.0, r + two_pi, r)
        ry = jnp.where(ry > np.pi, ry - two_pi, ry)

        s0 = start + 1 + 2 * _NUM_HEAD_BIN
        h = ld(s0 + 0) * _MEAN_SIZE[0] + _MEAN_SIZE[0]
        w = ld(s0 + 1) * _MEAN_SIZE[1] + _MEAN_SIZE[1]
        l = ld(s0 + 2) * _MEAN_SIZE[2] + _MEAN_SIZE[2]

        pos_x = pos_x + cx
        pos_z = pos_z + cz
        half_w = h / 2.0
        half_h = l / 2.0

        cols = [pos_x, pos_y, pos_z, h, w, l, ry, zero16,
                pos_x - half_w, pos_z - half_h, pos_x + half_w,
                pos_z + half_h, zero16, zero16, zero16, zero16]
        for c, vec in enumerate(cols):
            out_v[pl.ds(c * rows_per_tile + g * 16, 16)] = vec
        return carry

    lax.fori_loop(0, rows_per_tile // 16, g_body, 0)
    pltpu.sync_copy(out_v, out_hbm.at[wid])


def _sc_decode(reg_p, xyz_p):
    R = reg_p.shape[0]
    rows_per_tile = R // _NTILES
    mesh = plsc.VectorSubcoreMesh(core_axis_name="c", subcore_axis_name="s")
    k = pl.kernel(
        functools.partial(_sc_decode_body, rows_per_tile=rows_per_tile),
        out_type=jax.ShapeDtypeStruct((_NTILES, 16 * rows_per_tile),
                                      jnp.float32),
        mesh=mesh,
        scratch_types=[
            pltpu.VMEM((rows_per_tile * _PITCH,), jnp.float32),
            pltpu.VMEM((rows_per_tile * _XPITCH,), jnp.float32),
            pltpu.VMEM((16 * rows_per_tile,), jnp.float32),
        ],
        compiler_params=pltpu.CompilerParams(needs_layout_passes=False),
    )
    return k(reg_p.reshape(-1), xyz_p.reshape(-1))


def _nms_body(x1_ref, z1_ref, x2_ref, z2_ref, sc_ref, prop_ref,
              out_ref, *, n_valid, nb):
    x1 = x1_ref[...]
    z1 = z1_ref[...]
    x2 = x2_ref[...]
    z2 = z2_ref[...]
    sc = sc_ref[...]
    V = x1.shape[0]
    iv = jax.lax.broadcasted_iota(jnp.int32, x1.shape, 0)
    isub = jax.lax.broadcasted_iota(jnp.int32, x1.shape, 1)
    il = jax.lax.broadcasted_iota(jnp.int32, x1.shape, 2)
    sh = 8 // nb
    pos = iv * (sh * 128) + (isub // nb) * 128 + il
    areas = (x2 - x1) * (z2 - z1)
    lane8 = jax.lax.broadcasted_iota(jnp.int32, (1, 8), 1)

    def shared_tree(vals, op):
        vals = list(vals)
        while len(vals) > 1:
            nxt = [op(vals[i], vals[i + 1]) for i in range(0, len(vals) - 1, 2)]
            if len(vals) % 2:
                nxt.append(vals[-1])
            vals = nxt
        r = vals[0]
        r = op(r, pltpu.roll(r, nb, 0))
        if op is jnp.maximum:
            return jnp.max(r, axis=1, keepdims=True)
        return jnp.min(r, axis=1, keepdims=True)

    def write_rows(k, icol, mcol):
        for b in range(nb):
            m_b = mcol[b, 0]
            valid_b = m_b > _NEG
            idx_b = jnp.where(valid_b, icol[b, 0], 0)
            vf = valid_b.astype(jnp.float32)
            prow = prop_ref[b, pl.ds(idx_b, 1), :]
            row = (prow + jnp.where(lane8 == 7, m_b, 0.0)) * vf
            out_ref[b, pl.ds(k, 1), :] = row

    def body(k, carry):
        alive, picol, pmcol = carry
        kk = jnp.maximum(k - 1, 0)
        write_rows(kk, picol, pmcol)

        masked = jnp.where(alive > 0.0, sc, _NEG)
        mvs = [masked[i] for i in range(V)]
        mcol = shared_tree(mvs, jnp.maximum)
        cand = jnp.where(masked == mcol[None], pos, 1 << 30)
        icol = shared_tree([cand[i] for i in range(V)], jnp.minimum)
        validc = mcol > _NEG

        sel = pos == icol[None]
        x1c = shared_tree([jnp.where(sel[i], x1[i], _NEG) for i in range(V)],
                          jnp.maximum)
        z1c = shared_tree([jnp.where(sel[i], z1[i], _NEG) for i in range(V)],
                          jnp.maximum)
        x2c = shared_tree([jnp.where(sel[i], x2[i], _NEG) for i in range(V)],
                          jnp.maximum)
        z2c = shared_tree([jnp.where(sel[i], z2[i], _NEG) for i in range(V)],
                          jnp.maximum)
        areac = (x2c - x1c) * (z2c - z1c)

        xx1 = jnp.maximum(x1c[None], x1)
        zz1 = jnp.maximum(z1c[None], z1)
        xx2 = jnp.minimum(x2c[None], x2)
        zz2 = jnp.minimum(z2c[None], z2)
        inter = jnp.maximum(xx2 - xx1, 0.0) * jnp.maximum(zz2 - zz1, 0.0)
        iou = inter / jnp.maximum(areac[None] + areas - inter, 1e-8)
        sup = (iou > _NMS_THRES) | sel
        alive = jnp.where(validc[None] & sup, 0.0, alive)
        return alive, icol, mcol

    alive0 = (pos < n_valid).astype(jnp.float32)
    icol0 = jnp.zeros((8, 1), jnp.int32)
    mcol0 = jnp.full((8, 1), _NEG, jnp.float32)
    _, icol_f, mcol_f = jax.lax.fori_loop(
        0, _NMS_POST, body, (alive0, icol0, mcol0))
    write_rows(_NMS_POST - 1, icol_f, mcol_f)


def kernel(rpn_scores, rpn_reg, xyz):
    B, N = rpn_scores.shape
    C = rpn_reg.shape[-1]
    sh = 8 // B
    NP = ((N + sh * 128 - 1) // (sh * 128)) * (sh * 128)
    V = NP // (sh * 128)

    reg_p = jnp.pad(rpn_reg,
                    ((0, 0), (0, NP - N), (0, _PITCH - C))).reshape(-1, _PITCH)
    xyz_p = jnp.pad(xyz,
                    ((0, 0), (0, NP - N), (0, _XPITCH - 3))).reshape(-1, _XPITCH)
    dec = _sc_decode(reg_p, xyz_p)
    dec = dec.reshape(_NTILES, 16, -1)
    flat = dec.transpose(0, 2, 1).reshape(B, NP, 16)
    prop_p = flat[..., 0:8]

    def interleave(a):
        return (a.reshape(B, V, sh, 128)
                 .transpose(1, 2, 0, 3)
                 .reshape(V, 8, 128))

    x1 = interleave(flat[..., 8])
    z1 = interleave(flat[..., 9])
    x2 = interleave(flat[..., 10])
    z2 = interleave(flat[..., 11])
    scp = interleave(jnp.pad(rpn_scores, ((0, 0), (0, NP - N))))

    out = pl.pallas_call(
        functools.partial(_nms_body, n_valid=N, nb=B),
        out_shape=jax.ShapeDtypeStruct((B, _NMS_POST, 8), jnp.float32),
    )(x1, z1, x2, z2, scp, prop_p)

    return out[..., :7], out[..., 7]
